# baseline (speedup 1.0000x reference)
import functools

import jax
import jax.numpy as jnp
from jax import lax
from jax.experimental import pallas as pl
from jax.experimental.pallas import tpu as pltpu
from jax.experimental.pallas import tpu_sc as plsc

N = 8
C = 192
H = 56
W = 56
E = 16
D = 64
K = 4
S = H * W
S_BLK = 1568

SC_N = 2
NW = 32
ST8 = S // 8
SC_TILES = SC_N * ST8
SC_ITERS = (SC_TILES + NW - 1) // NW


def _gate_kernel(x_ref, w_ref, b_ref, pm_ref):
    x = x_ref[0]
    logits = jax.lax.dot_general(
        w_ref[...], x, (((1,), (1,)), ((), ())),
        preferred_element_type=jnp.float32) + b_ref[...]
    m = jnp.max(logits, axis=0, keepdims=True)
    p = jnp.exp(logits - m)
    probs = p / jnp.sum(p, axis=0, keepdims=True)

    iota_e = jax.lax.broadcasted_iota(jnp.int32, probs.shape, 0)
    mask = jnp.zeros(probs.shape, jnp.bool_)
    wp = probs
    for _ in range(K):
        mx = jnp.max(wp, axis=0, keepdims=True)
        sel_idx = jnp.min(jnp.where(wp == mx, iota_e, E), axis=0,
                          keepdims=True)
        sel = iota_e == sel_idx
        mask = jnp.logical_or(mask, sel)
        wp = jnp.where(sel, -jnp.inf, wp)
    pm_ref[0] = jnp.where(mask, probs, 0.0)


def _sum_kernel(pm_ref, ex_ref, out_ref):
    pm = pm_ref[0]
    acc = pm[:, 0:1] * ex_ref[0, 0]
    for e in range(1, E):
        acc = acc + pm[:, e:e + 1] * ex_ref[0, e]
    out_ref[0] = acc


def _bcast_lane(v, lane):
    idx = jnp.full((16, 1), lane, jnp.int32)
    return lax.gather(
        v, idx,
        dimension_numbers=lax.GatherDimensionNumbers(
            offset_dims=(), collapsed_slice_dims=(0,),
            start_index_map=(0,)),
        slice_sizes=(1,),
        mode=lax.GatherScatterMode.PROMISE_IN_BOUNDS)


_MESH = plsc.VectorSubcoreMesh(core_axis_name="c", subcore_axis_name="s")


@functools.partial(
    pl.kernel,
    mesh=_MESH,
    out_type=jax.ShapeDtypeStruct((SC_N * S * D,), jnp.float32),
    scratch_types=[
        pltpu.VMEM((E, 8, D), jnp.float32),
        pltpu.VMEM((8 * E,), jnp.float32),
        pltpu.VMEM((8 * D,), jnp.float32),
        pltpu.SemaphoreType.DMA,
    ],
    compiler_params=pltpu.CompilerParams(use_tc_tiling_on_sc=True),
)
def _sc_dense(ex_hbm, pm_hbm, out_hbm, ebuf, pmbuf, obuf, esem):
    wid = lax.axis_index("s") * 2 + lax.axis_index("c")

    def body(i, carry):
        t = wid + i * NW

        @pl.when(t < SC_TILES)
        def _():
            n = t // ST8
            s8 = t - n * ST8
            base_loc = n * S + s8 * 8
            for e in range(E):
                pltpu.async_copy(ex_hbm.at[n, e, s8], ebuf.at[e], esem)
            pltpu.sync_copy(pm_hbm.at[pl.ds(base_loc * E, 8 * E)], pmbuf)
            for e in range(E):
                pltpu.make_async_copy(ex_hbm.at[n, e, s8], ebuf.at[e],
                                      esem).wait()
            for i8 in range(8):
                pmv = pmbuf[pl.ds(E * i8, 16)]
                wbs = [_bcast_lane(pmv, e) for e in range(E)]
                for d in range(D // 16):
                    acc = wbs[0] * ebuf[0, i8, pl.ds(16 * d, 16)]
                    for e in range(1, E):
                        acc = acc + wbs[e] * ebuf[e, i8, pl.ds(16 * d, 16)]
                    obuf[pl.ds(i8 * D + 16 * d, 16)] = acc
            pltpu.sync_copy(obuf, out_hbm.at[pl.ds(base_loc * D, 8 * D)])
        return carry

    lax.fori_loop(0, SC_ITERS, body, 0)


@jax.jit
def kernel(x, experts, gate_w, gate_b):
    xs = jnp.transpose(x, (0, 2, 3, 1)).reshape(N, S, C)
    exs = experts.reshape(N, E, S, D)
    ex5 = experts.reshape(N, E, ST8, 8, D)
    b2 = gate_b.reshape(E, 1)

    pm0 = pl.pallas_call(
        _gate_kernel,
        grid=(N,),
        in_specs=[
            pl.BlockSpec((1, S, C), lambda n: (n, 0, 0)),
            pl.BlockSpec((E, C), lambda n: (0, 0)),
            pl.BlockSpec((E, 1), lambda n: (0, 0)),
        ],
        out_specs=pl.BlockSpec((1, E, S), lambda n: (n, 0, 0)),
        out_shape=jax.ShapeDtypeStruct((N, E, S), jnp.float32),
    )(xs, gate_w, b2)

    pmt = jnp.transpose(pm0, (0, 2, 1))

    pm_sc = pmt[:SC_N].reshape(SC_N * S * E)
    out_sc = _sc_dense(ex5, pm_sc)

    grid = (N - SC_N, S // S_BLK)
    out_tc = pl.pallas_call(
        _sum_kernel,
        grid=grid,
        in_specs=[
            pl.BlockSpec((1, S_BLK, E), lambda n, s: (n + SC_N, s, 0)),
            pl.BlockSpec((1, E, S_BLK, D), lambda n, s: (n + SC_N, 0, s, 0)),
        ],
        out_specs=pl.BlockSpec((1, S_BLK, D), lambda n, s: (n, s, 0)),
        out_shape=jax.ShapeDtypeStruct((N - SC_N, S, D), jnp.float32),
    )(pmt, exs)

    out = jnp.concatenate([out_sc.reshape(SC_N, S, D), out_tc], axis=0)
    return out.reshape(N, H, W, D)
